# baseline (speedup 1.0000x reference)
import functools

import jax
import jax.numpy as jnp
from jax import lax
from jax.experimental import pallas as pl
from jax.experimental.pallas import tpu as pltpu
from jax.experimental.pallas import tpu_sc as plsc

_W = 128
_CW = 128
_NC, _NS = 2, 16
_NW = _NC * _NS
_NB = 4
_C = 2


def _take16(vec, idx):
    dnums = lax.GatherDimensionNumbers(
        offset_dims=(), collapsed_slice_dims=(0,), start_index_map=(0,))
    return lax.gather(vec, idx[:, None], dnums, (1,),
                      mode=lax.GatherScatterMode.PROMISE_IN_BOUNDS)


def _tc_transpose(total, d, k):
    ntc = total // _W // _C
    bt = max(1, ntc // 4)
    nstep = ntc // bt

    def body(x_ref, o_ref):
        xb = x_ref[...].reshape(bt, _W, d)
        o_ref[...] = jnp.transpose(xb, (0, 2, 1)).reshape(bt * d, _W)

    return pl.pallas_call(
        body,
        grid=(nstep,),
        in_specs=[pl.BlockSpec((bt * _W, d), lambda i: (k * nstep + i, 0))],
        out_specs=pl.BlockSpec((bt * d, _W), lambda i: (i, 0)),
        out_shape=jax.ShapeDtypeStruct((ntc * d, _W), jnp.float32),
    )


def _sc_scatter(total, d, k):
    nchunk = d // _CW
    ntc = total // _W // _C
    per_w = ntc * nchunk // _NW

    mesh = plsc.VectorSubcoreMesh(core_axis_name="c", subcore_axis_name="s")

    @functools.partial(
        pl.kernel,
        out_type=(),
        mesh=mesh,
        compiler_params=pltpu.CompilerParams(needs_layout_passes=False,
                                             use_tc_tiling_on_sc=False),
        scratch_types=(
            [pltpu.VMEM((_CW, _W), jnp.float32) for _ in range(_NB)]
            + [pltpu.VMEM((1, _CW), jnp.int32) for _ in range(_NB)]
            + [pltpu.VMEM((16,), jnp.int32)]
            + [pltpu.SemaphoreType.DMA for _ in range(2 * _NB)]
        ),
    )
    def sc_kernel(stage_hbm, info_hbm, out_hbm, *refs):
        in_vs = refs[0:_NB]
        idx_vs = refs[_NB:2 * _NB]
        cu_v = refs[2 * _NB]
        in_ss = refs[2 * _NB + 1:3 * _NB + 1]
        sc_ss = refs[3 * _NB + 1:4 * _NB + 1]

        wid = lax.axis_index("s") * _NC + lax.axis_index("c")
        base_unit = wid * per_w
        iota = lax.iota(jnp.int32, 16)
        pltpu.sync_copy(info_hbm.at[pl.ds(0, 16)], cu_v)
        cu = cu_v[...]
        cu_next = jnp.where(iota == 15, jnp.int32(total),
                            _take16(cu, (iota + 1) & 15))

        def in_copy(u, b):
            t_loc = u // nchunk
            c0 = (u % nchunk) * _CW
            return pltpu.make_async_copy(
                stage_hbm.at[pl.ds(t_loc * d + c0, _CW)], in_vs[b], in_ss[b])

        def scat_copy(b):
            return pltpu.make_async_copy(
                in_vs[b], out_hbm.at[idx_vs[b].at[0]], sc_ss[b])

        in_copy(base_unit, 0).start()
        in_copy(base_unit + 1, 1).start()

        def outer(ii, carry):
            for b in range(_NB):
                u = base_unit + ii * _NB + b
                in_copy(u, b).wait()

                t = k * ntc + u // nchunk
                r0 = t * _W
                c0 = (u % nchunk) * _CW
                s = plsc.all_reduce_population_count(cu <= r0) - 1
                seg_base = _take16(cu, s)
                seg_end = _take16(cu_next, s)
                ldiv = (seg_end - seg_base) // _W
                base_off = (seg_base * (d // _W) + (r0 - seg_base) // _W
                            + c0 * ldiv)
                for kk in range(_CW // 16):
                    idx_vs[b][0, pl.ds(kk * 16, 16)] = (
                        base_off + (kk * 16 + iota) * ldiv)

                scat_copy(b).start()

                bj = (b + 2) % _NB
                j_ok = (ii * _NB + b + 2) < per_w
                if b >= 2:
                    @pl.when(j_ok)
                    def _():
                        scat_copy(bj).wait()
                        in_copy(u + 2, bj).start()
                else:
                    @pl.when(jnp.logical_and(ii > 0, j_ok))
                    def _():
                        scat_copy(bj).wait()
                        in_copy(u + 2, bj).start()

                    @pl.when(jnp.logical_and(ii == 0, j_ok))
                    def _():
                        in_copy(u + 2, bj).start()
            return carry

        lax.fori_loop(0, per_w // _NB, outer, 0, unroll=False)

        for b in range(_NB):
            scat_copy(b).wait()

    return sc_kernel


def kernel(x, info):
    total, d = x.shape
    out_ref = jax.ref.empty_ref(
        jax.ShapeDtypeStruct((total * d // _W, _W), jnp.float32))
    for k in range(_C):
        stage = _tc_transpose(total, d, k)(x)
        _sc_scatter(total, d, k)(stage, info, out_ref)
    out2d = jax.ref.freeze(out_ref)
    return jnp.reshape(out2d, (total * d,))
